# Initial kernel scaffold; baseline (speedup 1.0000x reference)
import jax
import jax.numpy as jnp
from jax.experimental import pallas as pl


def kernel(agent_pos, agent_context, ngh_pos, ngh_context, possible_lanes, lane_context, label, seq_start_end, valid_neighbor, W_msg, b_msg, W_ih, W_hh, b_ih, b_hh):
    raise NotImplementedError("write your pallas kernel here")


# speedup vs baseline: 82.3243x; 82.3243x over previous
import jax
import jax.numpy as jnp
from jax.experimental import pallas as pl


def _body(B, P, S, H, N,
          sse_ref, valid_ref, actx_ref, nctx_tab_ref, npx_tab_ref,
          npy_tab_ref, lx_ref, ly_ref, Wp_ref, Wn_ref, Wa_ref,
          Wih_ref, Whh_ref, bm_ref, bi_ref, bh_ref, out_ref):
    starts = sse_ref[:, 0:1]
    ends = sse_ref[:, 1:2]
    iota_n = jax.lax.broadcasted_iota(jnp.int32, (B, N), 1)
    onehot = (iota_n == starts).astype(jnp.float32)

    nctx = jnp.dot(onehot, nctx_tab_ref[...],
                   preferred_element_type=jnp.float32)
    npx = jnp.sum(onehot * npx_tab_ref[...], axis=1, keepdims=True)
    npy = jnp.sum(onehot * npy_tab_ref[...], axis=1, keepdims=True)

    actx = actx_ref[...]
    xg = (jnp.dot(nctx, Wn_ref[...], preferred_element_type=jnp.float32)
          + jnp.dot(actx, Wa_ref[...], preferred_element_type=jnp.float32)
          + (-npx) * Wp_ref[0:1, :] + (-npy) * Wp_ref[1:2, :]
          + bm_ref[...])
    x = jnp.maximum(xg, 0.0)

    gi = jnp.dot(x, Wih_ref[...], preferred_element_type=jnp.float32) + bi_ref[...]
    gh = jnp.dot(nctx, Whh_ref[...], preferred_element_type=jnp.float32) + bh_ref[...]
    r_g = jax.nn.sigmoid(gi[:, :H] + gh[:, :H])
    z = jax.nn.sigmoid(gi[:, H:2 * H] + gh[:, H:2 * H])
    n_g = jnp.tanh(gi[:, 2 * H:] + r_g * gh[:, 2 * H:])
    r = (1.0 - z) * n_g + z * nctx

    cond = jnp.logical_and(valid_ref[...] > 0, (ends - starts) > 0)

    d2min = jnp.full((B, P), jnp.inf, jnp.float32)
    nan_any = jnp.zeros((B, P), jnp.bool_)
    for s in range(S):
        lxs = lx_ref[s]
        lys = ly_ref[s]
        nan_any = nan_any | jnp.isnan(lxs) | jnp.isnan(lys)
        dx = npx - lxs
        dy = npy - lys
        d2min = jnp.minimum(d2min, dx * dx + dy * dy)
    d2 = jnp.where(nan_any, npx * npx + npy * npy, d2min)
    keep = cond & (d2 < 10000.0)

    for l in range(P):
        out_ref[:, l, :] = jnp.where(keep[:, l:l + 1], r, 0.0)


def kernel(agent_pos, agent_context, ngh_pos, ngh_context, possible_lanes,
           lane_context, label, seq_start_end, valid_neighbor,
           W_msg, b_msg, W_ih, W_hh, b_ih, b_hh):
    B, P, H = lane_context.shape
    S = possible_lanes.shape[0]
    N = ngh_context.shape[0]

    lx = possible_lanes[:, :, 0].reshape(S, B, P)
    ly = possible_lanes[:, :, 1].reshape(S, B, P)
    npx_tab = ngh_pos[:, 0].reshape(1, N)
    npy_tab = ngh_pos[:, 1].reshape(1, N)
    Wp = W_msg[:, :2].T
    Wn = W_msg[:, 2:2 + H].T
    Wa = W_msg[:, 2 + H:].T
    Wih_t = W_ih.T
    Whh_t = W_hh.T
    bm2 = b_msg.reshape(1, H)
    bi2 = b_ih.reshape(1, 3 * H)
    bh2 = b_hh.reshape(1, 3 * H)
    valid_i = valid_neighbor.astype(jnp.int32).reshape(B, 1)

    import functools
    body = functools.partial(_body, B, P, S, H, N)
    out2 = pl.pallas_call(
        body,
        out_shape=jax.ShapeDtypeStruct((B, P, H), jnp.float32),
    )(seq_start_end, valid_i, agent_context, ngh_context, npx_tab, npy_tab,
      lx, ly, Wp, Wn, Wa, Wih_t, Whh_t, bm2, bi2, bh2)

    return (lane_context, out2)
